# Initial kernel scaffold; baseline (speedup 1.0000x reference)
import jax
import jax.numpy as jnp
from jax.experimental import pallas as pl


def kernel(input):
    raise NotImplementedError("write your pallas kernel here")


# speedup vs baseline: 58.7678x; 58.7678x over previous
import jax
import jax.numpy as jnp
from jax.experimental import pallas as pl


def _mode_body(x_ref, bg_ref, fg_ref):
    x = x_ref[...]
    B = x.shape[0]
    cnt = jnp.zeros_like(x)
    for j in range(B):
        cnt = cnt + jnp.where(x == x[j:j + 1, :], 1.0, 0.0)
    key = cnt * 256.0 + (255.0 - x)
    best = jnp.max(key, axis=0, keepdims=True)
    rem = best - 256.0 * jnp.floor(best * (1.0 / 256.0))
    mode = 255.0 - rem
    bg = jnp.broadcast_to(mode, x.shape)
    bg_ref[...] = bg
    fg_ref[...] = jnp.abs(x - bg)


def kernel(input):
    B, C, H, W = input.shape
    N = C * H * W
    x2 = input.reshape(B, N)
    BLK = 512
    grid = (N // BLK,)
    bg, fg = pl.pallas_call(
        _mode_body,
        grid=grid,
        in_specs=[pl.BlockSpec((B, BLK), lambda i: (0, i))],
        out_specs=[pl.BlockSpec((B, BLK), lambda i: (0, i)),
                   pl.BlockSpec((B, BLK), lambda i: (0, i))],
        out_shape=[jax.ShapeDtypeStruct((B, N), jnp.float32),
                   jax.ShapeDtypeStruct((B, N), jnp.float32)],
    )(x2)
    return bg.reshape(input.shape), fg.reshape(input.shape)
